# baseline (speedup 1.0000x reference)
import jax
import jax.numpy as jnp
from jax.experimental import pallas as pl

N_TOK = 8192
N_EXP = 64
K = 8
BLOCK = 8192
NEG_INF = float("-inf")


def _gate_kernel(x_ref, w_ref, out_w_ref, out_i_ref):
    x = x_ref[...]
    w = w_ref[...]
    lt = jax.lax.dot_general(
        w, x, (((1,), (1,)), ((), ())), preferred_element_type=jnp.float32
    )
    eidx = jax.lax.broadcasted_iota(jnp.int32, lt.shape, 0)
    kiota = jax.lax.broadcasted_iota(jnp.int32, (K, BLOCK), 0)

    l = lt
    vals = jnp.zeros((K, BLOCK), jnp.float32)
    idxs = jnp.zeros((K, BLOCK), jnp.int32)
    m = None
    for k in range(K):
        cur = jnp.max(l, axis=0, keepdims=True)
        idx = jnp.min(jnp.where(l == cur, eidx, N_EXP), axis=0, keepdims=True)
        vals = jnp.where(kiota == k, cur, vals)
        idxs = jnp.where(kiota == k, idx, idxs)
        if k == 0:
            m = cur
            s = jnp.sum(jnp.exp(lt - m), axis=0, keepdims=True)
        if k + 1 < K:
            l = jnp.where(eidx == idx, NEG_INF, l)

    wts = jnp.exp(vals - m) / s
    out_w_ref[...] = wts.T
    out_i_ref[...] = idxs.T


@jax.jit
def kernel(hidden_states, weight):
    grid = (N_TOK // BLOCK,)
    out_w, out_i = pl.pallas_call(
        _gate_kernel,
        grid=grid,
        in_specs=[
            pl.BlockSpec((BLOCK, N_EXP), lambda i: (i, 0)),
            pl.BlockSpec((N_EXP, N_EXP), lambda i: (0, 0)),
        ],
        out_specs=[
            pl.BlockSpec((BLOCK, K), lambda i: (i, 0)),
            pl.BlockSpec((BLOCK, K), lambda i: (i, 0)),
        ],
        out_shape=[
            jax.ShapeDtypeStruct((N_TOK, K), jnp.float32),
            jax.ShapeDtypeStruct((N_TOK, K), jnp.int32),
        ],
    )(hidden_states, weight)
    return (out_w, out_i)


# speedup vs baseline: 1.0274x; 1.0274x over previous
import jax
import jax.numpy as jnp
from jax.experimental import pallas as pl

N_TOK = 8192
N_EXP = 64
K = 8
BLOCK = 4096
NEG_INF = float("-inf")


def _gate_kernel(x_ref, w_ref, out_w_ref, out_i_ref):
    x = x_ref[...]
    w = w_ref[...]
    lt = jax.lax.dot_general(
        w, x, (((1,), (1,)), ((), ())), preferred_element_type=jnp.float32
    )
    eidx = jax.lax.broadcasted_iota(jnp.int32, lt.shape, 0)
    kiota = jax.lax.broadcasted_iota(jnp.int32, (K, BLOCK), 0)

    l = lt
    vals = jnp.zeros((K, BLOCK), jnp.float32)
    idxs = jnp.zeros((K, BLOCK), jnp.int32)
    m = None
    for k in range(K):
        cur = jnp.max(l, axis=0, keepdims=True)
        idx = jnp.min(jnp.where(l == cur, eidx, N_EXP), axis=0, keepdims=True)
        vals = jnp.where(kiota == k, cur, vals)
        idxs = jnp.where(kiota == k, idx, idxs)
        if k == 0:
            m = cur
            s = jnp.sum(jnp.exp(lt - m), axis=0, keepdims=True)
        if k + 1 < K:
            l = jnp.where(eidx == idx, NEG_INF, l)

    wts = jnp.exp(vals - m) / s
    out_w_ref[...] = wts.T
    out_i_ref[...] = idxs.T


@jax.jit
def kernel(hidden_states, weight):
    grid = (N_TOK // BLOCK,)
    out_w, out_i = pl.pallas_call(
        _gate_kernel,
        grid=grid,
        in_specs=[
            pl.BlockSpec((BLOCK, N_EXP), lambda i: (i, 0)),
            pl.BlockSpec((N_EXP, N_EXP), lambda i: (0, 0)),
        ],
        out_specs=[
            pl.BlockSpec((BLOCK, K), lambda i: (i, 0)),
            pl.BlockSpec((BLOCK, K), lambda i: (i, 0)),
        ],
        out_shape=[
            jax.ShapeDtypeStruct((N_TOK, K), jnp.float32),
            jax.ShapeDtypeStruct((N_TOK, K), jnp.int32),
        ],
    )(hidden_states, weight)
    return (out_w, out_i)
